# Initial kernel scaffold; baseline (speedup 1.0000x reference)
import jax
import jax.numpy as jnp
from jax.experimental import pallas as pl


def kernel(x, edge_index, W1, b1, W2, b2, W3, b3):
    raise NotImplementedError("write your pallas kernel here")


# speedup vs baseline: 5.6583x; 5.6583x over previous
import functools
import jax
import jax.numpy as jnp
from jax import lax
from jax.experimental import pallas as pl
from jax.experimental.pallas import tpu as pltpu
from jax.experimental.pallas import tpu_sc as plsc

N = 10000
NPAD = 10240
E = 320000
NC = 2
NS = 16
DH = 128
HN = NPAD // 2
TRASH = 256
AROWS = HN + TRASH
CHUNK = 80
BLKC = 25
NBLK = (E // NS) // (BLKC * CHUNK)
NBLK2 = (E // (NC * NS)) // (BLKC * CHUNK)
RPT = AROWS // NS
WPT = HN // NS
ZROWS = 64

_mesh = plsc.VectorSubcoreMesh(
    core_axis_name="c", subcore_axis_name="s", num_cores=NC, num_subcores=NS
)


def _fill_vmem(ref, rows, width, value):
    v = jnp.full((16,), value, jnp.float32)

    def body(r, carry):
        for k in range(width // 16):
            ref[r, pl.ds(k * 16, 16)] = v
        return carry

    lax.fori_loop(0, rows, body, 0, unroll=False)


@functools.partial(
    pl.kernel,
    out_type=jax.ShapeDtypeStruct((NC, NPAD, DH), jnp.float32),
    mesh=_mesh,
    scratch_types=[
        pltpu.VMEM((NBLK * BLKC, CHUNK), jnp.int32),
        pltpu.VMEM((NBLK * BLKC, CHUNK), jnp.int32),
        pltpu.VMEM((CHUNK, DH), jnp.float32),
        pltpu.VMEM((ZROWS, DH), jnp.float32),
        pltpu.VMEM_SHARED((AROWS, DH), jnp.float32),
        pltpu.SemaphoreType.DMA,
    ],
)
def _scatter_kernel(g_hbm, srcc_hbm, dstm_hbm, out_hbm,
                    src_v, dstm_v, rows_v, zbuf, acc, sem):
    c = lax.axis_index("c")
    s = lax.axis_index("s")

    _fill_vmem(zbuf, ZROWS, DH, 0.0)

    for r in range(2):
        zb = s * RPT
        for i in range(RPT // ZROWS):
            pltpu.sync_copy(zbuf, acc.at[pl.ds(zb + i * ZROWS, ZROWS)])
        pltpu.sync_copy(zbuf.at[pl.ds(0, RPT % ZROWS)],
                        acc.at[pl.ds(zb + (RPT // ZROWS) * ZROWS, RPT % ZROWS)])
        plsc.subcore_barrier()

        lo = r * HN
        pltpu.sync_copy(srcc_hbm.at[c, s], src_v)
        pltpu.sync_copy(dstm_hbm.at[r, s], dstm_v)

        def body(j, carry):
            pltpu.async_copy(g_hbm.at[src_v.at[j]], rows_v, sem).wait()
            pltpu.sync_copy(rows_v, acc.at[dstm_v.at[j]], add=True)
            return carry

        lax.fori_loop(0, NBLK * BLKC, body, 0, unroll=False)
        plsc.subcore_barrier()

        wb = s * WPT
        for i in range(WPT // ZROWS):
            pltpu.sync_copy(acc.at[pl.ds(wb + i * ZROWS, ZROWS)], zbuf)
            pltpu.sync_copy(
                zbuf, out_hbm.at[c, pl.ds(lo + wb + i * ZROWS, ZROWS)])
        plsc.subcore_barrier()
        _fill_vmem(zbuf, ZROWS, DH, 0.0)


@functools.partial(
    pl.kernel,
    out_type=jax.ShapeDtypeStruct((NC, NPAD, 16), jnp.float32),
    mesh=_mesh,
    scratch_types=[
        pltpu.VMEM((NBLK2 * BLKC, CHUNK), jnp.int32),
        pltpu.VMEM((CHUNK, 16), jnp.float32),
        pltpu.VMEM((128, 16), jnp.float32),
        pltpu.VMEM_SHARED((NPAD, 16), jnp.float32),
    ],
)
def _degree_kernel(dst_hbm, out_hbm, dst_v, ones_v, zbuf, acc):
    c = lax.axis_index("c")
    s = lax.axis_index("s")
    base = s * (NPAD // NS)

    _fill_vmem(zbuf, 128, 16, 0.0)
    _fill_vmem(ones_v, CHUNK, 16, 1.0)
    for i in range(NPAD // NS // 128):
        pltpu.sync_copy(zbuf, acc.at[pl.ds(base + i * 128, 128)])

    pltpu.sync_copy(dst_hbm.at[c, s], dst_v)
    plsc.subcore_barrier()

    def body(j, carry):
        pltpu.sync_copy(ones_v, acc.at[dst_v.at[j]], add=True)
        return carry

    lax.fori_loop(0, NBLK2 * BLKC, body, 0, unroll=False)
    plsc.subcore_barrier()

    for i in range(NPAD // NS // 128):
        r0 = base + i * 128
        pltpu.sync_copy(acc.at[pl.ds(r0, 128)], zbuf)
        pltpu.sync_copy(zbuf, out_hbm.at[c, pl.ds(r0, 128)])


def _dis_of(cnt_blk):
    deg = cnt_blk[0, :, 0:1] + cnt_blk[1, :, 0:1] + 1.0
    return lax.rsqrt(deg)


ROWS_B = 400


def _t1_body(x_ref, w1_ref, cnt_ref, out_ref):
    dis = _dis_of(cnt_ref[...])
    y = jnp.dot(x_ref[...], w1_ref[...], preferred_element_type=jnp.float32)
    out_ref[0] = dis * y[:, :128]
    out_ref[1] = dis * y[:, 128:]


def _t_mid_body(s_ref, g_ref, b_ref, w_ref, cnt_ref, out_ref):
    dis = _dis_of(cnt_ref[...])
    ua = jax.nn.relu(dis * (s_ref[0] + g_ref[0]) + b_ref[0, 0][None, :])
    ub = jax.nn.relu(dis * (s_ref[1] + g_ref[1]) + b_ref[0, 1][None, :])
    y = (jnp.dot(ua, w_ref[0], preferred_element_type=jnp.float32)
         + jnp.dot(ub, w_ref[1], preferred_element_type=jnp.float32))
    out_ref[0] = dis * y[:, :128]
    out_ref[1] = dis * y[:, 128:]


def _t4_body(s_ref, g_ref, b_ref, cnt_ref, out_ref):
    dis = _dis_of(cnt_ref[...])
    out_ref[:, pl.ds(0, 64)] = (
        dis * (s_ref[0, :, :64] + g_ref[0, :, :64]) + b_ref[0, 0][None, :])
    out_ref[:, pl.ds(64, 64)] = (
        dis * (s_ref[1, :, :64] + g_ref[1, :, :64]) + b_ref[0, 1][None, :])


def _t1_call(x, w1, cnt2):
    return pl.pallas_call(
        _t1_body,
        grid=(N // ROWS_B,),
        in_specs=[
            pl.BlockSpec((ROWS_B, 128), lambda i: (i, 0)),
            pl.BlockSpec((128, 256), lambda i: (0, 0)),
            pl.BlockSpec((2, ROWS_B, 16), lambda i: (0, i, 0)),
        ],
        out_specs=pl.BlockSpec((2, ROWS_B, 128), lambda i: (0, i, 0)),
        out_shape=jax.ShapeDtypeStruct((2, NPAD, 128), jnp.float32),
    )(x, w1, cnt2)


def _t_mid_call(s_in, g_in, b_in, w_in, cnt2):
    return pl.pallas_call(
        _t_mid_body,
        grid=(N // ROWS_B,),
        in_specs=[
            pl.BlockSpec((2, ROWS_B, 128), lambda i: (0, i, 0)),
            pl.BlockSpec((2, ROWS_B, 128), lambda i: (0, i, 0)),
            pl.BlockSpec((1, 2, 128), lambda i: (0, 0, 0)),
            pl.BlockSpec((2, 128, 256), lambda i: (0, 0, 0)),
            pl.BlockSpec((2, ROWS_B, 16), lambda i: (0, i, 0)),
        ],
        out_specs=pl.BlockSpec((2, ROWS_B, 128), lambda i: (0, i, 0)),
        out_shape=jax.ShapeDtypeStruct((2, NPAD, 128), jnp.float32),
    )(s_in, g_in, b_in, w_in, cnt2)


def _t4_call(s3, g3, b3, cnt2):
    return pl.pallas_call(
        _t4_body,
        grid=(N // ROWS_B,),
        in_specs=[
            pl.BlockSpec((2, ROWS_B, 128), lambda i: (0, i, 0)),
            pl.BlockSpec((2, ROWS_B, 128), lambda i: (0, i, 0)),
            pl.BlockSpec((1, 2, 64), lambda i: (0, 0, 0)),
            pl.BlockSpec((2, ROWS_B, 16), lambda i: (0, i, 0)),
        ],
        out_specs=pl.BlockSpec((ROWS_B, 128), lambda i: (i, 0)),
        out_shape=jax.ShapeDtypeStruct((N, 128), jnp.float32),
    )(s3, g3, b3, cnt2)


def kernel(x, edge_index, W1, b1, W2, b2, W3, b3):
    src = edge_index[0]
    dst = edge_index[1]
    srcc = jnp.stack([src, src + NPAD]).reshape(NC, NS, NBLK * BLKC, CHUNK)
    trash = HN + (dst & (TRASH - 1))
    dstm = jnp.stack([
        jnp.where(dst < HN, dst, trash),
        jnp.where(dst >= HN, dst - HN, trash),
    ]).reshape(2, NS, NBLK * BLKC, CHUNK)
    dst_half = dst.reshape(NC, NS, NBLK2 * BLKC, CHUNK)

    w2r = W2.reshape(2, 128, 256)
    w3p = (jnp.zeros((256, 256), jnp.float32)
           .at[:, 0:64].set(W3[:, 0:64])
           .at[:, 128:192].set(W3[:, 64:128]))
    w3r = w3p.reshape(2, 128, 256)
    b1r = b1.reshape(1, 2, 128)
    b2r = b2.reshape(1, 2, 128)
    b3r = b3.reshape(1, 2, 64)

    wstack = jnp.stack([w2r, w3r, w3r])
    bstack = jnp.stack([b1r, b2r, b2r])

    cnt2 = _degree_kernel(dst_half)
    g1 = _t1_call(x, W1, cnt2)

    def body(carry, wb):
        g_cur, _, _ = carry
        w, b = wb
        s_cur = _scatter_kernel(g_cur.reshape(2 * NPAD, 128), srcc, dstm)
        g_next = _t_mid_call(s_cur, g_cur, b, w, cnt2)
        return (g_next, g_cur, s_cur), None

    init = (g1, g1, jnp.zeros((NC, NPAD, 128), jnp.float32))
    (_, g3, s3), _ = lax.scan(body, init, (wstack, bstack))
    return _t4_call(s3, g3, b3r, cnt2)


# speedup vs baseline: 6.4324x; 1.1368x over previous
import functools
import jax
import jax.numpy as jnp
from jax import lax
from jax.experimental import pallas as pl
from jax.experimental.pallas import tpu as pltpu
from jax.experimental.pallas import tpu_sc as plsc

N = 10000
NPAD = 10240
E = 320000
NC = 2
NS = 16
DH = 128
HN = NPAD // 2
TRASH = 128
AROWS = HN + TRASH
CHUNK = 125
BLKC = 16
NBLK = (E // NS) // (BLKC * CHUNK)
NBLK2 = (E // (NC * NS)) // (BLKC * CHUNK)
RPT = AROWS // NS
WPT = HN // NS
ZROWS = 64

_mesh = plsc.VectorSubcoreMesh(
    core_axis_name="c", subcore_axis_name="s", num_cores=NC, num_subcores=NS
)


def _fill_vmem(ref, rows, width, value):
    v = jnp.full((16,), value, jnp.float32)

    def body(r, carry):
        for k in range(width // 16):
            ref[r, pl.ds(k * 16, 16)] = v
        return carry

    lax.fori_loop(0, rows, body, 0, unroll=False)


@functools.partial(
    pl.kernel,
    out_type=jax.ShapeDtypeStruct((NC, NPAD, DH), jnp.float32),
    mesh=_mesh,
    scratch_types=[
        pltpu.VMEM((NBLK * BLKC, CHUNK), jnp.int32),
        pltpu.VMEM((NBLK * BLKC, CHUNK), jnp.int32),
        pltpu.VMEM((CHUNK, DH), jnp.float32),
        pltpu.VMEM((ZROWS, DH), jnp.float32),
        pltpu.VMEM_SHARED((AROWS, DH), jnp.float32),
        pltpu.SemaphoreType.DMA,
    ],
)
def _scatter_kernel(g_hbm, srcc_hbm, dstm_hbm, out_hbm,
                    src_v, dstm_v, rows_a, zbuf, acc, sem_a):
    c = lax.axis_index("c")
    s = lax.axis_index("s")

    _fill_vmem(zbuf, ZROWS, DH, 0.0)

    for r in range(2):
        zb = s * RPT
        for i in range(RPT // ZROWS):
            pltpu.sync_copy(zbuf, acc.at[pl.ds(zb + i * ZROWS, ZROWS)])
        pltpu.sync_copy(zbuf.at[pl.ds(0, RPT % ZROWS)],
                        acc.at[pl.ds(zb + (RPT // ZROWS) * ZROWS, RPT % ZROWS)])
        plsc.subcore_barrier()

        lo = r * HN
        pltpu.sync_copy(srcc_hbm.at[c, s], src_v)
        pltpu.sync_copy(dstm_hbm.at[r, s], dstm_v)

        def body(j, carry):
            pltpu.async_copy(g_hbm.at[src_v.at[j]], rows_a, sem_a).wait()
            pltpu.sync_copy(rows_a, acc.at[dstm_v.at[j]], add=True)
            return carry

        lax.fori_loop(0, NBLK * BLKC, body, 0, unroll=False)
        plsc.subcore_barrier()

        wb = s * WPT
        for i in range(WPT // ZROWS):
            pltpu.sync_copy(acc.at[pl.ds(wb + i * ZROWS, ZROWS)], zbuf)
            pltpu.sync_copy(
                zbuf, out_hbm.at[c, pl.ds(lo + wb + i * ZROWS, ZROWS)])
        plsc.subcore_barrier()
        _fill_vmem(zbuf, ZROWS, DH, 0.0)


QN = NPAD // 4
DAROWS = QN + TRASH


@functools.partial(
    pl.kernel,
    out_type=jax.ShapeDtypeStruct((NC, NPAD, 16), jnp.float32),
    mesh=_mesh,
    scratch_types=[
        pltpu.VMEM((NBLK2 * BLKC, CHUNK), jnp.int32),
        pltpu.VMEM((CHUNK, 16), jnp.float32),
        pltpu.VMEM((128, 16), jnp.float32),
        pltpu.VMEM_SHARED((DAROWS, 16), jnp.float32),
    ],
)
def _degree_kernel(dstq_hbm, out_hbm, dst_v, ones_v, zbuf, acc):
    c = lax.axis_index("c")
    s = lax.axis_index("s")
    w = c * NS + s

    _fill_vmem(zbuf, 128, 16, 0.0)
    _fill_vmem(ones_v, CHUNK, 16, 1.0)

    for r in range(4):
        zb = s * (DAROWS // NS)
        pltpu.sync_copy(zbuf, acc.at[pl.ds(zb, 128)])
        pltpu.sync_copy(zbuf.at[pl.ds(0, 40)], acc.at[pl.ds(zb + 128, 40)])
        pltpu.sync_copy(dstq_hbm.at[r, w], dst_v)
        plsc.subcore_barrier()

        def body(j, carry):
            pltpu.sync_copy(ones_v, acc.at[dst_v.at[j]], add=True)
            return carry

        lax.fori_loop(0, NBLK2 * BLKC, body, 0, unroll=False)
        plsc.subcore_barrier()

        wb = s * (QN // NS)
        pltpu.sync_copy(acc.at[pl.ds(wb, 128)], zbuf)
        pltpu.sync_copy(zbuf, out_hbm.at[c, pl.ds(r * QN + wb, 128)])
        pltpu.sync_copy(acc.at[pl.ds(wb + 128, 32)], zbuf.at[pl.ds(0, 32)])
        pltpu.sync_copy(zbuf.at[pl.ds(0, 32)],
                        out_hbm.at[c, pl.ds(r * QN + wb + 128, 32)])
        plsc.subcore_barrier()


def _dis_of(cnt_blk):
    deg = cnt_blk[0, :, 0:1] + cnt_blk[1, :, 0:1] + 1.0
    return lax.rsqrt(deg)


ROWS_B = 400


def _t1_body(x_ref, w1_ref, cnt_ref, out_ref):
    dis = _dis_of(cnt_ref[...])
    y = jnp.dot(x_ref[...], w1_ref[...], preferred_element_type=jnp.float32)
    out_ref[0] = dis * y[:, :128]
    out_ref[1] = dis * y[:, 128:]


def _t_mid_body(s_ref, g_ref, b_ref, w_ref, cnt_ref, out_ref):
    dis = _dis_of(cnt_ref[...])
    ua = jax.nn.relu(dis * (s_ref[0] + g_ref[0]) + b_ref[0, 0][None, :])
    ub = jax.nn.relu(dis * (s_ref[1] + g_ref[1]) + b_ref[0, 1][None, :])
    y = (jnp.dot(ua, w_ref[0], preferred_element_type=jnp.float32)
         + jnp.dot(ub, w_ref[1], preferred_element_type=jnp.float32))
    out_ref[0] = dis * y[:, :128]
    out_ref[1] = dis * y[:, 128:]


def _t4_body(s_ref, g_ref, b_ref, cnt_ref, out_ref):
    dis = _dis_of(cnt_ref[...])
    out_ref[:, pl.ds(0, 64)] = (
        dis * (s_ref[0, :, :64] + g_ref[0, :, :64]) + b_ref[0, 0][None, :])
    out_ref[:, pl.ds(64, 64)] = (
        dis * (s_ref[1, :, :64] + g_ref[1, :, :64]) + b_ref[0, 1][None, :])


def _t1_call(x, w1, cnt2):
    return pl.pallas_call(
        _t1_body,
        grid=(N // ROWS_B,),
        in_specs=[
            pl.BlockSpec((ROWS_B, 128), lambda i: (i, 0)),
            pl.BlockSpec((128, 256), lambda i: (0, 0)),
            pl.BlockSpec((2, ROWS_B, 16), lambda i: (0, i, 0)),
        ],
        out_specs=pl.BlockSpec((2, ROWS_B, 128), lambda i: (0, i, 0)),
        out_shape=jax.ShapeDtypeStruct((2, NPAD, 128), jnp.float32),
    )(x, w1, cnt2)


def _t_mid_call(s_in, g_in, b_in, w_in, cnt2):
    return pl.pallas_call(
        _t_mid_body,
        grid=(N // ROWS_B,),
        in_specs=[
            pl.BlockSpec((2, ROWS_B, 128), lambda i: (0, i, 0)),
            pl.BlockSpec((2, ROWS_B, 128), lambda i: (0, i, 0)),
            pl.BlockSpec((1, 2, 128), lambda i: (0, 0, 0)),
            pl.BlockSpec((2, 128, 256), lambda i: (0, 0, 0)),
            pl.BlockSpec((2, ROWS_B, 16), lambda i: (0, i, 0)),
        ],
        out_specs=pl.BlockSpec((2, ROWS_B, 128), lambda i: (0, i, 0)),
        out_shape=jax.ShapeDtypeStruct((2, NPAD, 128), jnp.float32),
    )(s_in, g_in, b_in, w_in, cnt2)


def _t4_call(s3, g3, b3, cnt2):
    return pl.pallas_call(
        _t4_body,
        grid=(N // ROWS_B,),
        in_specs=[
            pl.BlockSpec((2, ROWS_B, 128), lambda i: (0, i, 0)),
            pl.BlockSpec((2, ROWS_B, 128), lambda i: (0, i, 0)),
            pl.BlockSpec((1, 2, 64), lambda i: (0, 0, 0)),
            pl.BlockSpec((2, ROWS_B, 16), lambda i: (0, i, 0)),
        ],
        out_specs=pl.BlockSpec((ROWS_B, 128), lambda i: (i, 0)),
        out_shape=jax.ShapeDtypeStruct((N, 128), jnp.float32),
    )(s3, g3, b3, cnt2)


def kernel(x, edge_index, W1, b1, W2, b2, W3, b3):
    src = edge_index[0]
    dst = edge_index[1]
    srcc = jnp.stack([src, src + NPAD]).reshape(NC, NS, NBLK * BLKC, CHUNK)
    trash = HN + (dst & (TRASH - 1))
    dstm = jnp.stack([
        jnp.where(dst < HN, dst, trash),
        jnp.where(dst >= HN, dst - HN, trash),
    ]).reshape(2, NS, NBLK * BLKC, CHUNK)
    trashq = QN + (dst & (TRASH - 1))
    dstq = jnp.stack([
        jnp.where((dst >= r * QN) & (dst < (r + 1) * QN), dst - r * QN, trashq)
        for r in range(4)
    ]).reshape(4, NC * NS, NBLK2 * BLKC, CHUNK)

    w2r = W2.reshape(2, 128, 256)
    w3p = (jnp.zeros((256, 256), jnp.float32)
           .at[:, 0:64].set(W3[:, 0:64])
           .at[:, 128:192].set(W3[:, 64:128]))
    w3r = w3p.reshape(2, 128, 256)
    b1r = b1.reshape(1, 2, 128)
    b2r = b2.reshape(1, 2, 128)
    b3r = b3.reshape(1, 2, 64)

    wstack = jnp.stack([w2r, w3r, w3r])
    bstack = jnp.stack([b1r, b2r, b2r])

    cnt2 = _degree_kernel(dstq)
    g1 = _t1_call(x, W1, cnt2)

    def body(carry, wb):
        g_cur, _, _ = carry
        w, b = wb
        s_cur = _scatter_kernel(g_cur.reshape(2 * NPAD, 128), srcc, dstm)
        g_next = _t_mid_call(s_cur, g_cur, b, w, cnt2)
        return (g_next, g_cur, s_cur), None

    init = (g1, g1, jnp.zeros((NC, NPAD, 128), jnp.float32))
    (_, g3, s3), _ = lax.scan(body, init, (wstack, bstack))
    return _t4_call(s3, g3, b3r, cnt2)


# speedup vs baseline: 7.5364x; 1.1716x over previous
import functools
import jax
import jax.numpy as jnp
from jax import lax
from jax.experimental import pallas as pl
from jax.experimental.pallas import tpu as pltpu
from jax.experimental.pallas import tpu_sc as plsc

N = 10000
NPAD = 10240
E = 320000
NC = 2
NS = 16
DH = 128
HN = NPAD // 2
TRASH = 128
AROWS = HN + TRASH
CHUNK = 125
BLKC = 16
NBLK = (E // NS) // (BLKC * CHUNK)
NBLK2 = (E // (NC * NS)) // (BLKC * CHUNK)
RPT = AROWS // NS
WPT = HN // NS
ZROWS = 64

_mesh = plsc.VectorSubcoreMesh(
    core_axis_name="c", subcore_axis_name="s", num_cores=NC, num_subcores=NS
)


def _fill_vmem(ref, rows, width, value):
    v = jnp.full((16,), value, jnp.float32)

    def body(r, carry):
        for k in range(width // 16):
            ref[r, pl.ds(k * 16, 16)] = v
        return carry

    lax.fori_loop(0, rows, body, 0, unroll=False)


@functools.partial(
    pl.kernel,
    out_type=jax.ShapeDtypeStruct((NC, NPAD, DH), jnp.float32),
    mesh=_mesh,
    scratch_types=[
        pltpu.VMEM((NBLK * BLKC, CHUNK), jnp.int32),
        pltpu.VMEM((NBLK * BLKC, CHUNK), jnp.int32),
        pltpu.VMEM((CHUNK, DH), jnp.float32),
        pltpu.VMEM((CHUNK, DH), jnp.float32),
        pltpu.VMEM((ZROWS, DH), jnp.float32),
        pltpu.VMEM_SHARED((AROWS, DH), jnp.float32),
        pltpu.SemaphoreType.DMA,
        pltpu.SemaphoreType.DMA,
    ],
)
def _scatter_kernel(g_hbm, srcc_hbm, dstm_hbm, out_hbm,
                    src_v, dstm_v, rows_a, rows_b, zbuf, acc, sem_a, sem_b):
    c = lax.axis_index("c")
    s = lax.axis_index("s")

    _fill_vmem(zbuf, ZROWS, DH, 0.0)

    for r in range(2):
        zb = s * RPT
        for i in range(RPT // ZROWS):
            pltpu.sync_copy(zbuf, acc.at[pl.ds(zb + i * ZROWS, ZROWS)])
        pltpu.sync_copy(zbuf.at[pl.ds(0, RPT % ZROWS)],
                        acc.at[pl.ds(zb + (RPT // ZROWS) * ZROWS, RPT % ZROWS)])
        plsc.subcore_barrier()

        lo = r * HN
        pltpu.sync_copy(srcc_hbm.at[c, s], src_v)
        pltpu.sync_copy(dstm_hbm.at[r, s], dstm_v)

        def body(j2, carry):
            ja = 2 * j2
            da = pltpu.async_copy(g_hbm.at[src_v.at[ja]], rows_a, sem_a)
            db = pltpu.async_copy(g_hbm.at[src_v.at[ja + 1]], rows_b, sem_b)
            da.wait()
            pltpu.sync_copy(rows_a, acc.at[dstm_v.at[ja]], add=True)
            db.wait()
            pltpu.sync_copy(rows_b, acc.at[dstm_v.at[ja + 1]], add=True)
            return carry

        lax.fori_loop(0, (NBLK * BLKC) // 2, body, 0, unroll=False)
        plsc.subcore_barrier()

        wb = s * WPT
        for i in range(WPT // ZROWS):
            pltpu.sync_copy(acc.at[pl.ds(wb + i * ZROWS, ZROWS)], zbuf)
            pltpu.sync_copy(
                zbuf, out_hbm.at[c, pl.ds(lo + wb + i * ZROWS, ZROWS)])
        plsc.subcore_barrier()
        _fill_vmem(zbuf, ZROWS, DH, 0.0)


QN = NPAD // 4
DAROWS = QN + TRASH


@functools.partial(
    pl.kernel,
    out_type=jax.ShapeDtypeStruct((NC, NPAD, 16), jnp.float32),
    mesh=_mesh,
    scratch_types=[
        pltpu.VMEM((NBLK2 * BLKC, CHUNK), jnp.int32),
        pltpu.VMEM((CHUNK, 16), jnp.float32),
        pltpu.VMEM((128, 16), jnp.float32),
        pltpu.VMEM_SHARED((DAROWS, 16), jnp.float32),
    ],
)
def _degree_kernel(dstq_hbm, out_hbm, dst_v, ones_v, zbuf, acc):
    c = lax.axis_index("c")
    s = lax.axis_index("s")
    w = c * NS + s

    _fill_vmem(zbuf, 128, 16, 0.0)
    _fill_vmem(ones_v, CHUNK, 16, 1.0)

    for r in range(4):
        zb = s * (DAROWS // NS)
        pltpu.sync_copy(zbuf, acc.at[pl.ds(zb, 128)])
        pltpu.sync_copy(zbuf.at[pl.ds(0, 40)], acc.at[pl.ds(zb + 128, 40)])
        pltpu.sync_copy(dstq_hbm.at[r, w], dst_v)
        plsc.subcore_barrier()

        def body(j, carry):
            pltpu.sync_copy(ones_v, acc.at[dst_v.at[j]], add=True)
            return carry

        lax.fori_loop(0, NBLK2 * BLKC, body, 0, unroll=False)
        plsc.subcore_barrier()

        wb = s * (QN // NS)
        pltpu.sync_copy(acc.at[pl.ds(wb, 128)], zbuf)
        pltpu.sync_copy(zbuf, out_hbm.at[c, pl.ds(r * QN + wb, 128)])
        pltpu.sync_copy(acc.at[pl.ds(wb + 128, 32)], zbuf.at[pl.ds(0, 32)])
        pltpu.sync_copy(zbuf.at[pl.ds(0, 32)],
                        out_hbm.at[c, pl.ds(r * QN + wb + 128, 32)])
        plsc.subcore_barrier()


def _dis_of(cnt_blk):
    deg = cnt_blk[0, :, 0:1] + cnt_blk[1, :, 0:1] + 1.0
    return lax.rsqrt(deg)


ROWS_B = 400


def _t1_body(x_ref, w1_ref, cnt_ref, out_ref):
    dis = _dis_of(cnt_ref[...])
    y = jnp.dot(x_ref[...], w1_ref[...], preferred_element_type=jnp.float32)
    out_ref[0] = dis * y[:, :128]
    out_ref[1] = dis * y[:, 128:]


def _t_mid_body(s_ref, g_ref, b_ref, w_ref, cnt_ref, out_ref):
    dis = _dis_of(cnt_ref[...])
    ua = jax.nn.relu(dis * (s_ref[0] + g_ref[0]) + b_ref[0, 0][None, :])
    ub = jax.nn.relu(dis * (s_ref[1] + g_ref[1]) + b_ref[0, 1][None, :])
    y = (jnp.dot(ua, w_ref[0], preferred_element_type=jnp.float32)
         + jnp.dot(ub, w_ref[1], preferred_element_type=jnp.float32))
    out_ref[0] = dis * y[:, :128]
    out_ref[1] = dis * y[:, 128:]


def _t4_body(s_ref, g_ref, b_ref, cnt_ref, out_ref):
    dis = _dis_of(cnt_ref[...])
    out_ref[:, pl.ds(0, 64)] = (
        dis * (s_ref[0, :, :64] + g_ref[0, :, :64]) + b_ref[0, 0][None, :])
    out_ref[:, pl.ds(64, 64)] = (
        dis * (s_ref[1, :, :64] + g_ref[1, :, :64]) + b_ref[0, 1][None, :])


def _t1_call(x, w1, cnt2):
    return pl.pallas_call(
        _t1_body,
        grid=(N // ROWS_B,),
        in_specs=[
            pl.BlockSpec((ROWS_B, 128), lambda i: (i, 0)),
            pl.BlockSpec((128, 256), lambda i: (0, 0)),
            pl.BlockSpec((2, ROWS_B, 16), lambda i: (0, i, 0)),
        ],
        out_specs=pl.BlockSpec((2, ROWS_B, 128), lambda i: (0, i, 0)),
        out_shape=jax.ShapeDtypeStruct((2, NPAD, 128), jnp.float32),
    )(x, w1, cnt2)


def _t_mid_call(s_in, g_in, b_in, w_in, cnt2):
    return pl.pallas_call(
        _t_mid_body,
        grid=(N // ROWS_B,),
        in_specs=[
            pl.BlockSpec((2, ROWS_B, 128), lambda i: (0, i, 0)),
            pl.BlockSpec((2, ROWS_B, 128), lambda i: (0, i, 0)),
            pl.BlockSpec((1, 2, 128), lambda i: (0, 0, 0)),
            pl.BlockSpec((2, 128, 256), lambda i: (0, 0, 0)),
            pl.BlockSpec((2, ROWS_B, 16), lambda i: (0, i, 0)),
        ],
        out_specs=pl.BlockSpec((2, ROWS_B, 128), lambda i: (0, i, 0)),
        out_shape=jax.ShapeDtypeStruct((2, NPAD, 128), jnp.float32),
    )(s_in, g_in, b_in, w_in, cnt2)


def _t4_call(s3, g3, b3, cnt2):
    return pl.pallas_call(
        _t4_body,
        grid=(N // ROWS_B,),
        in_specs=[
            pl.BlockSpec((2, ROWS_B, 128), lambda i: (0, i, 0)),
            pl.BlockSpec((2, ROWS_B, 128), lambda i: (0, i, 0)),
            pl.BlockSpec((1, 2, 64), lambda i: (0, 0, 0)),
            pl.BlockSpec((2, ROWS_B, 16), lambda i: (0, i, 0)),
        ],
        out_specs=pl.BlockSpec((ROWS_B, 128), lambda i: (i, 0)),
        out_shape=jax.ShapeDtypeStruct((N, 128), jnp.float32),
    )(s3, g3, b3, cnt2)


def kernel(x, edge_index, W1, b1, W2, b2, W3, b3):
    src = edge_index[0]
    dst = edge_index[1]
    srcc = jnp.stack([src, src + NPAD]).reshape(NC, NS, NBLK * BLKC, CHUNK)
    trash = HN + (dst & (TRASH - 1))
    dstm = jnp.stack([
        jnp.where(dst < HN, dst, trash),
        jnp.where(dst >= HN, dst - HN, trash),
    ]).reshape(2, NS, NBLK * BLKC, CHUNK)
    trashq = QN + (dst & (TRASH - 1))
    dstq = jnp.stack([
        jnp.where((dst >= r * QN) & (dst < (r + 1) * QN), dst - r * QN, trashq)
        for r in range(4)
    ]).reshape(4, NC * NS, NBLK2 * BLKC, CHUNK)

    w2r = W2.reshape(2, 128, 256)
    w3p = (jnp.zeros((256, 256), jnp.float32)
           .at[:, 0:64].set(W3[:, 0:64])
           .at[:, 128:192].set(W3[:, 64:128]))
    w3r = w3p.reshape(2, 128, 256)
    b1r = b1.reshape(1, 2, 128)
    b2r = b2.reshape(1, 2, 128)
    b3r = b3.reshape(1, 2, 64)

    wstack = jnp.stack([w2r, w3r, w3r])
    bstack = jnp.stack([b1r, b2r, b2r])

    cnt2 = _degree_kernel(dstq)
    g1 = _t1_call(x, W1, cnt2)

    def body(carry, wb):
        g_cur, _, _ = carry
        w, b = wb
        s_cur = _scatter_kernel(g_cur.reshape(2 * NPAD, 128), srcc, dstm)
        g_next = _t_mid_call(s_cur, g_cur, b, w, cnt2)
        return (g_next, g_cur, s_cur), None

    init = (g1, g1, jnp.zeros((NC, NPAD, 128), jnp.float32))
    (_, g3, s3), _ = lax.scan(body, init, (wstack, bstack))
    return _t4_call(s3, g3, b3r, cnt2)
